# baseline (speedup 1.0000x reference)
import functools

import jax
import jax.numpy as jnp
from jax import lax
from jax.experimental import pallas as pl
from jax.experimental.pallas import tpu as pltpu
from jax.experimental.pallas import tpu_sc as plsc

NG = 64
KNN = 16
EPS = 1e-5
RB = 256
CB = 256
EB = 128
KRB = 128

_pallas_call = pl.pallas_call


def _colstats_kernel(n_valid, x_ref, st_ref):
    b = pl.program_id(0)

    @pl.when(b == 0)
    def _():
        st_ref[...] = jnp.zeros_like(st_ref)

    xb = x_ref[...]
    rid = lax.broadcasted_iota(jnp.int32, (1, xb.shape[0]), 1) + b * xb.shape[0]
    m = (rid < n_valid).astype(xb.dtype)
    st_ref[0:1, :] += jnp.dot(m, xb, preferred_element_type=jnp.float32)
    st_ref[1:2, :] += jnp.dot(m, xb * xb, preferred_element_type=jnp.float32)


def _knn_kernel(bounds_ref, pts_ref, ptsT_ref, brow_ref, bcol_ref, nbr_ref):
    b = pl.program_id(0)
    r0 = b * KRB
    rows = pts_ref[pl.ds(r0, KRB), :]
    batch_r = brow_ref[pl.ds(r0, KRB), :]
    row_ids = lax.broadcasted_iota(jnp.int32, (KRB, 1), 0) + r0
    lo = bounds_ref[b, 0]
    hi = bounds_ref[b, 1]
    inf = jnp.float32(jnp.inf)
    ibig = jnp.int32(2 ** 30)

    def tile_body(t, carry):
        bv, bi = carry
        c0 = t * CB
        colsT = ptsT_ref[:, pl.ds(c0, CB)]
        batch_c = bcol_ref[:, pl.ds(c0, CB)]
        colsq = jnp.sum(colsT * colsT, axis=0, keepdims=True)
        score = colsq - 2.0 * jnp.dot(rows, colsT,
                                      preferred_element_type=jnp.float32)
        col_ids = lax.broadcasted_iota(jnp.int32, (KRB, CB), 1) + c0
        valid = (batch_r == batch_c) & (col_ids != row_ids)
        score = jnp.where(valid, score, inf)
        comb_v = jnp.concatenate([bv, score], axis=1)
        comb_i = jnp.concatenate([bi, col_ids], axis=1)
        nv, ni = [], []
        for _ in range(KNN):
            m = jnp.min(comb_v, axis=1, keepdims=True)
            ismin = comb_v == m
            sel = jnp.min(jnp.where(ismin, comb_i, ibig), axis=1,
                          keepdims=True)
            nv.append(m)
            ni.append(sel)
            comb_v = jnp.where(ismin & (comb_i == sel), inf, comb_v)
        return (jnp.concatenate(nv, axis=1), jnp.concatenate(ni, axis=1))

    init = (jnp.full((KRB, KNN), inf, jnp.float32),
            jnp.full((KRB, KNN), ibig, jnp.int32))
    _, bi = lax.fori_loop(lo, hi, tile_body, init)
    nbr_ref[...] = bi


def _pre_kernel(n_valid, f_ref, sc_ref, sh_ref, w1d_ref, w1b_ref, ws_ref,
                p_ref, q_ref, s_ref, st_ref):
    b = pl.program_id(0)

    @pl.when(b == 0)
    def _():
        st_ref[...] = jnp.zeros_like(st_ref)

    t = f_ref[...] * sc_ref[...] + sh_ref[...]
    p_ref[...] = jnp.dot(t, w1d_ref[...], preferred_element_type=jnp.float32)
    q_ref[...] = jnp.dot(t, w1b_ref[...], preferred_element_type=jnp.float32)
    s = jnp.dot(t, ws_ref[...], preferred_element_type=jnp.float32)
    s_ref[...] = s
    rid = lax.broadcasted_iota(jnp.int32, (1, s.shape[0]), 1) + b * s.shape[0]
    m = (rid < n_valid).astype(s.dtype)
    st_ref[0:1, :] += jnp.dot(m, s, preferred_element_type=jnp.float32)
    st_ref[1:2, :] += jnp.dot(m, s * s, preferred_element_type=jnp.float32)


def _edge_stats_kernel(n_valid, qg_ref, p_ref, st_ref):
    b = pl.program_id(0)

    @pl.when(b == 0)
    def _():
        st_ref[...] = jnp.zeros_like(st_ref)

    pb = p_ref[...]
    rid = lax.broadcasted_iota(jnp.int32, (1, pb.shape[0]), 1) + b * pb.shape[0]
    m = (rid < n_valid).astype(jnp.float32)
    sa = jnp.zeros((1, pb.shape[1]), jnp.float32)
    qa = jnp.zeros((1, pb.shape[1]), jnp.float32)
    for j in range(KNN):
        y = qg_ref[j] + pb
        sa = sa + jnp.dot(m, y, preferred_element_type=jnp.float32)
        qa = qa + jnp.dot(m, y * y, preferred_element_type=jnp.float32)
    st_ref[0:1, :] += sa
    st_ref[1:2, :] += qa


def _edge_mm1_kernel(n_valid, qg_ref, p_ref, mu_ref, w_ref, out_ref, st_ref):
    b = pl.program_id(0)

    @pl.when(b == 0)
    def _():
        st_ref[...] = jnp.zeros_like(st_ref)

    pb = p_ref[...] - mu_ref[...]
    rid = lax.broadcasted_iota(jnp.int32, (1, pb.shape[0]), 1) + b * pb.shape[0]
    m = (rid < n_valid).astype(jnp.float32)
    sa = jnp.zeros((1, w_ref.shape[1]), jnp.float32)
    qa = jnp.zeros((1, w_ref.shape[1]), jnp.float32)
    for j in range(KNN):
        h = jnp.maximum(qg_ref[j] + pb, 0.0)
        y2 = jnp.dot(h, w_ref[...], preferred_element_type=jnp.float32)
        out_ref[j] = y2
        sa = sa + jnp.dot(m, y2, preferred_element_type=jnp.float32)
        qa = qa + jnp.dot(m, y2 * y2, preferred_element_type=jnp.float32)
    st_ref[0:1, :] += sa
    st_ref[1:2, :] += qa


def _edge_mm2_kernel(n_valid, y_ref, mu_ref, w_ref, out_ref, st_ref):
    b = pl.program_id(0)

    @pl.when(b == 0)
    def _():
        st_ref[...] = jnp.zeros_like(st_ref)

    nrow = y_ref.shape[1]
    rid = lax.broadcasted_iota(jnp.int32, (1, nrow), 1) + b * nrow
    m = (rid < n_valid).astype(jnp.float32)
    mu = mu_ref[...]
    sa = jnp.zeros((1, w_ref.shape[1]), jnp.float32)
    qa = jnp.zeros((1, w_ref.shape[1]), jnp.float32)
    for j in range(KNN):
        h = jnp.maximum(y_ref[j] - mu, 0.0)
        y2 = jnp.dot(h, w_ref[...], preferred_element_type=jnp.float32)
        out_ref[j] = y2
        sa = sa + jnp.dot(m, y2, preferred_element_type=jnp.float32)
        qa = qa + jnp.dot(m, y2 * y2, preferred_element_type=jnp.float32)
    st_ref[0:1, :] += sa
    st_ref[1:2, :] += qa


def _aggr_kernel(y3_ref, mu3_ref, sc3_ref, s_ref, scs_ref, shs_ref, o_ref):
    mu3 = mu3_ref[...]
    acc = jnp.maximum(y3_ref[0] - mu3, 0.0)
    for j in range(1, KNN):
        acc = jnp.maximum(acc, y3_ref[j] - mu3)
    acc = jnp.maximum(acc, 0.0) * sc3_ref[...]
    skip = s_ref[...] * scs_ref[...] + shs_ref[...]
    o_ref[...] = jnp.maximum(acc + skip, 0.0)


def _pool_kernel(f_ref, bcol_ref, o_ref):
    b = pl.program_id(0)

    @pl.when(b == 0)
    def _():
        o_ref[...] = jnp.zeros_like(o_ref)

    fb = f_ref[...]
    bt = bcol_ref[...]
    seg = lax.broadcasted_iota(jnp.int32, (NG, fb.shape[0]), 0)
    onehot = (seg == bt).astype(jnp.float32)
    o_ref[...] += jnp.dot(onehot, fb, preferred_element_type=jnp.float32)


def _head_kernel(ncls, ps_ref, ic_ref, fcw_ref, fcb_ref, ow_ref, ob_ref,
                 o_ref):
    pooled = ps_ref[...] * ic_ref[...]
    h = jnp.maximum(
        jnp.dot(pooled, fcw_ref[...], preferred_element_type=jnp.float32)
        + fcb_ref[...], 0.0)
    lg = jnp.dot(h, ow_ref[...], preferred_element_type=jnp.float32) \
        + ob_ref[...]
    lane = lax.broadcasted_iota(jnp.int32, lg.shape, 1)
    lg = jnp.where(lane < ncls, lg, -jnp.inf)
    mx = jnp.max(lg, axis=1, keepdims=True)
    e = jnp.exp(lg - mx)
    o_ref[...] = e / jnp.sum(e, axis=1, keepdims=True)


def _gather_rows(q, idx):
    e = idx.shape[0]
    c = q.shape[1]
    info = plsc.get_sparse_core_info()
    nw = info.num_cores * info.num_subcores
    chunk = 128
    per_w = e // nw
    nch = per_w // chunk
    mesh = plsc.VectorSubcoreMesh(core_axis_name="c", subcore_axis_name="s")

    @functools.partial(
        pl.kernel,
        out_type=jax.ShapeDtypeStruct((e, c), jnp.float32),
        mesh=mesh,
        scratch_types=[
            pltpu.VMEM((chunk,), jnp.int32),
            pltpu.VMEM((chunk,), jnp.int32),
            pltpu.VMEM((chunk, c), jnp.float32),
            pltpu.VMEM((chunk, c), jnp.float32),
            pltpu.SemaphoreType.DMA,
            pltpu.SemaphoreType.DMA,
            pltpu.SemaphoreType.DMA,
            pltpu.SemaphoreType.DMA,
        ],
    )
    def gk(q_hbm, idx_hbm, out_hbm, i0, i1, r0, r1, g0, g1, s0, s1):
        wid = lax.axis_index("s") * info.num_cores + lax.axis_index("c")
        base = wid * per_w
        idx_v = (i0, i1)
        rows_v = (r0, r1)
        gsem = (g0, g1)
        ssem = (s0, s1)

        def body(t2, carry):
            offs = [base + (t2 * 2 + b) * chunk for b in range(2)]
            gets = []
            for b in range(2):
                pltpu.sync_copy(idx_hbm.at[pl.ds(offs[b], chunk)], idx_v[b])
                gets.append(pltpu.async_copy(q_hbm.at[idx_v[b]], rows_v[b],
                                             gsem[b]))
            puts = []
            for b in range(2):
                gets[b].wait()
                puts.append(pltpu.async_copy(
                    rows_v[b], out_hbm.at[pl.ds(offs[b], chunk)], ssem[b]))
            for b in range(2):
                puts[b].wait()
            return carry

        lax.fori_loop(0, nch // 2, body, 0)

    return gk(q, idx)


def _bn_coeffs(st, g, b, cnt):
    s = st[0:1, :]
    ss = st[1:2, :]
    mu = s / cnt
    var = ss / cnt - mu * mu
    inv = g.reshape(1, -1) / jnp.sqrt(var + EPS)
    return inv, b.reshape(1, -1) - mu * inv


def _bn_mu_inv(st, g, b, cnt):
    mu = st[0:1, :] / cnt
    var = st[1:2, :] / cnt - mu * mu
    inv = g.reshape(1, -1) / jnp.sqrt(var + EPS)
    return mu - b.reshape(1, -1) / inv, inv


def _colstats(xp, n):
    npad, d = xp.shape
    return _pallas_call(
        functools.partial(_colstats_kernel, n),
        grid=(npad // RB,),
        in_specs=[pl.BlockSpec((RB, d), lambda i: (i, 0))],
        out_specs=pl.BlockSpec((2, d), lambda i: (0, 0)),
        out_shape=jax.ShapeDtypeStruct((2, d), jnp.float32),
    )(xp)


def _knn(bounds, pts, ptsT, brow, bcol):
    npad, d = pts.shape
    return _pallas_call(
        _knn_kernel,
        grid=(npad // KRB,),
        in_specs=[
            pl.BlockSpec(memory_space=pltpu.SMEM),
            pl.BlockSpec((npad, d), lambda i: (0, 0)),
            pl.BlockSpec((d, npad), lambda i: (0, 0)),
            pl.BlockSpec((npad, 1), lambda i: (0, 0)),
            pl.BlockSpec((1, npad), lambda i: (0, 0)),
        ],
        out_specs=pl.BlockSpec((KRB, KNN), lambda i: (i, 0)),
        out_shape=jax.ShapeDtypeStruct((npad, KNN), jnp.int32),
    )(bounds, pts, ptsT, brow, bcol)


def _pre(fts, sc0, sh0, w1d, w1b, ws, n):
    npad, d = fts.shape
    c = w1d.shape[1]
    cq = w1b.shape[1]
    cs = ws.shape[1]
    return _pallas_call(
        functools.partial(_pre_kernel, n),
        grid=(npad // RB,),
        in_specs=[
            pl.BlockSpec((RB, d), lambda i: (i, 0)),
            pl.BlockSpec((1, d), lambda i: (0, 0)),
            pl.BlockSpec((1, d), lambda i: (0, 0)),
            pl.BlockSpec((d, c), lambda i: (0, 0)),
            pl.BlockSpec((d, cq), lambda i: (0, 0)),
            pl.BlockSpec((d, cs), lambda i: (0, 0)),
        ],
        out_specs=[
            pl.BlockSpec((RB, c), lambda i: (i, 0)),
            pl.BlockSpec((RB, cq), lambda i: (i, 0)),
            pl.BlockSpec((RB, cs), lambda i: (i, 0)),
            pl.BlockSpec((2, cs), lambda i: (0, 0)),
        ],
        out_shape=[
            jax.ShapeDtypeStruct((npad, c), jnp.float32),
            jax.ShapeDtypeStruct((npad, cq), jnp.float32),
            jax.ShapeDtypeStruct((npad, cs), jnp.float32),
            jax.ShapeDtypeStruct((2, cs), jnp.float32),
        ],
    )(fts, sc0, sh0, w1d, w1b, ws)


def _edge_stats(qg3, p, n):
    _, npad, _ = qg3.shape
    c = p.shape[1]
    return _pallas_call(
        functools.partial(_edge_stats_kernel, n),
        grid=(npad // EB,),
        in_specs=[
            pl.BlockSpec((KNN, EB, c), lambda i: (0, i, 0)),
            pl.BlockSpec((EB, c), lambda i: (i, 0)),
        ],
        out_specs=pl.BlockSpec((2, c), lambda i: (0, 0)),
        out_shape=jax.ShapeDtypeStruct((2, c), jnp.float32),
    )(qg3, p)


def _edge_mm1(qg3, p, mu, w, n):
    _, npad, _ = qg3.shape
    c = p.shape[1]
    c2 = w.shape[1]
    return _pallas_call(
        functools.partial(_edge_mm1_kernel, n),
        grid=(npad // EB,),
        in_specs=[
            pl.BlockSpec((KNN, EB, c), lambda i: (0, i, 0)),
            pl.BlockSpec((EB, c), lambda i: (i, 0)),
            pl.BlockSpec((1, c), lambda i: (0, 0)),
            pl.BlockSpec((c, c2), lambda i: (0, 0)),
        ],
        out_specs=[
            pl.BlockSpec((KNN, EB, c2), lambda i: (0, i, 0)),
            pl.BlockSpec((2, c2), lambda i: (0, 0)),
        ],
        out_shape=[
            jax.ShapeDtypeStruct((KNN, npad, c2), jnp.float32),
            jax.ShapeDtypeStruct((2, c2), jnp.float32),
        ],
    )(qg3, p, mu, w)


def _edge_mm2(y, mu, w, n):
    _, npad, c = y.shape
    c2 = w.shape[1]
    return _pallas_call(
        functools.partial(_edge_mm2_kernel, n),
        grid=(npad // EB,),
        in_specs=[
            pl.BlockSpec((KNN, EB, c), lambda i: (0, i, 0)),
            pl.BlockSpec((1, c), lambda i: (0, 0)),
            pl.BlockSpec((c, c2), lambda i: (0, 0)),
        ],
        out_specs=[
            pl.BlockSpec((KNN, EB, c2), lambda i: (0, i, 0)),
            pl.BlockSpec((2, c2), lambda i: (0, 0)),
        ],
        out_shape=[
            jax.ShapeDtypeStruct((KNN, npad, c2), jnp.float32),
            jax.ShapeDtypeStruct((2, c2), jnp.float32),
        ],
    )(y, mu, w)


def _aggr(y3, mu3, sc3, s, scs, shs):
    _, npad, c = y3.shape
    return _pallas_call(
        _aggr_kernel,
        grid=(npad // RB,),
        in_specs=[
            pl.BlockSpec((KNN, RB, c), lambda i: (0, i, 0)),
            pl.BlockSpec((1, c), lambda i: (0, 0)),
            pl.BlockSpec((1, c), lambda i: (0, 0)),
            pl.BlockSpec((RB, c), lambda i: (i, 0)),
            pl.BlockSpec((1, c), lambda i: (0, 0)),
            pl.BlockSpec((1, c), lambda i: (0, 0)),
        ],
        out_specs=pl.BlockSpec((RB, c), lambda i: (i, 0)),
        out_shape=jax.ShapeDtypeStruct((npad, c), jnp.float32),
    )(y3, mu3, sc3, s, scs, shs)


def _pool(fts, bcol):
    npad, c = fts.shape
    return _pallas_call(
        _pool_kernel,
        grid=(npad // RB,),
        in_specs=[
            pl.BlockSpec((RB, c), lambda i: (i, 0)),
            pl.BlockSpec((1, RB), lambda i: (0, i)),
        ],
        out_specs=pl.BlockSpec((NG, c), lambda i: (0, 0)),
        out_shape=jax.ShapeDtypeStruct((NG, c), jnp.float32),
    )(fts, bcol)


def _head(ps, ic, fcw, fcb, ow, ob, ncls):
    c = ps.shape[1]
    cf = fcw.shape[1]
    co = ow.shape[1]
    return _pallas_call(
        functools.partial(_head_kernel, ncls),
        in_specs=[
            pl.BlockSpec((NG, c), lambda: (0, 0)),
            pl.BlockSpec((NG, 1), lambda: (0, 0)),
            pl.BlockSpec((c, cf), lambda: (0, 0)),
            pl.BlockSpec((1, cf), lambda: (0, 0)),
            pl.BlockSpec((cf, co), lambda: (0, 0)),
            pl.BlockSpec((1, co), lambda: (0, 0)),
        ],
        out_specs=pl.BlockSpec((NG, co), lambda: (0, 0)),
        out_shape=jax.ShapeDtypeStruct((NG, co), jnp.float32),
    )(ps, ic, fcw, fcb, ow, ob)


def kernel(x, batch, params):
    n, din = x.shape
    npad = ((n + RB - 1) // RB) * RB
    pad = npad - n
    batch = batch.astype(jnp.int32)
    xp = jnp.pad(x, ((0, pad), (0, 0)))
    bp = jnp.pad(batch, (0, pad), constant_values=-1)
    brow = bp.reshape(npad, 1)
    bcol = bp.reshape(1, npad)

    gids = jnp.arange(NG, dtype=jnp.int32)
    seg_st = jnp.searchsorted(batch, gids, side='left').astype(jnp.int32)
    seg_en = jnp.searchsorted(batch, gids, side='right').astype(jnp.int32)

    nblk = npad // KRB
    first = jnp.minimum(jnp.arange(nblk, dtype=jnp.int32) * KRB, n - 1)
    last = jnp.minimum(first + KRB - 1, n - 1)
    cs = seg_st[batch[first]]
    ce = seg_en[batch[last]]
    bounds = jnp.stack([cs // CB, (ce + CB - 1) // CB], axis=1)

    st0 = _colstats(xp, n)
    scf, shf = _bn_coeffs(st0, params['bn_g'], params['bn_b'], float(n))

    fts = xp
    pts = jnp.pad(x[:, :3], ((0, pad), (0, 5)))
    ptsT = pts.T
    ecnt = float(n * KNN)

    for lp in params['convs']:
        d = fts.shape[1]
        w1 = lp['W1']
        w1d = (w1[:, :d] - w1[:, d:]).T
        w1b = w1[:, d:].T
        w2 = lp['W2'].T
        w3 = lp['W3'].T
        ws = lp['Ws'].T
        c = w1d.shape[1]

        cq = max(c, 128)
        w1dp = jnp.pad(w1d, ((0, 0), (0, cq - c)))
        w1bp = jnp.pad(w1b, ((0, 0), (0, cq - c)))
        w2p = jnp.pad(w2, ((0, cq - c), (0, 0)))

        nbr = _knn(bounds, pts, ptsT, brow, bcol)
        p, q, s, st_s = _pre(fts, scf, shf, w1dp, w1bp, ws, n)
        idx = jnp.where(nbr < n, nbr, 0).T.reshape(KNN * npad)
        qg3 = _gather_rows(q, idx).reshape(KNN, npad, cq)

        st1 = _edge_stats(qg3, p, n)
        mu1, inv1 = _bn_mu_inv(st1[:, :c], lp['g1'], lp['b1'], ecnt)
        mu1p = jnp.pad(mu1, ((0, 0), (0, cq - c)))
        w2f = w2p * jnp.pad(inv1, ((0, 0), (0, cq - c))).reshape(cq, 1)
        y2, st2 = _edge_mm1(qg3, p, mu1p, w2f, n)
        mu2, inv2 = _bn_mu_inv(st2, lp['g2'], lp['b2'], ecnt)
        w3f = w3 * inv2.reshape(-1, 1)
        y3, st3 = _edge_mm2(y2, mu2, w3f, n)
        mu3, inv3 = _bn_mu_inv(st3, lp['g3'], lp['b3'], ecnt)
        scs, shs = _bn_coeffs(st_s, lp['gs'], lp['bs'], float(n))
        fts = _aggr(y3, mu3, inv3, s, scs, shs)

        cn = fts.shape[1]
        scf = jnp.ones((1, cn), jnp.float32)
        shf = jnp.zeros((1, cn), jnp.float32)
        pts = fts
        ptsT = fts.T

    ps = _pool(fts, bcol)
    cnts = (seg_en - seg_st).astype(jnp.float32)
    ic = (1.0 / jnp.maximum(cnts, 1.0)).reshape(NG, 1)

    ncls = params['out_W'].shape[0]
    ow = jnp.pad(params['out_W'].T, ((0, 0), (0, 128 - ncls)))
    ob = jnp.pad(params['out_b'].reshape(1, -1), ((0, 0), (0, 128 - ncls)))
    probs = _head(ps, ic, params['fc_W'].T,
                  params['fc_b'].reshape(1, -1), ow, ob, ncls)
    return probs[:, :ncls]


# speedup vs baseline: 1.1425x; 1.1425x over previous
import functools

import jax
import jax.numpy as jnp
from jax import lax
from jax.experimental import pallas as pl
from jax.experimental.pallas import tpu as pltpu
from jax.experimental.pallas import tpu_sc as plsc

NG = 64
KNN = 16
EPS = 1e-5
RB = 256
CB = 256
EB = 128
KRB = 256

_pallas_call = pl.pallas_call


def _colstats_kernel(n_valid, x_ref, st_ref):
    b = pl.program_id(0)

    @pl.when(b == 0)
    def _():
        st_ref[...] = jnp.zeros_like(st_ref)

    xb = x_ref[...]
    rid = lax.broadcasted_iota(jnp.int32, (1, xb.shape[0]), 1) + b * xb.shape[0]
    m = (rid < n_valid).astype(xb.dtype)
    st_ref[0:1, :] += jnp.dot(m, xb, preferred_element_type=jnp.float32)
    st_ref[1:2, :] += jnp.dot(m, xb * xb, preferred_element_type=jnp.float32)


def _knn_kernel(bounds_ref, pts_ref, ptsT_ref, brow_ref, bcol_ref, nbr_ref):
    b = pl.program_id(0)
    r0 = b * KRB
    rows = pts_ref[pl.ds(r0, KRB), :]
    batch_r = brow_ref[pl.ds(r0, KRB), :]
    row_ids = lax.broadcasted_iota(jnp.int32, (KRB, 1), 0) + r0
    lo = bounds_ref[b, 0]
    hi = bounds_ref[b, 1]
    inf = jnp.float32(jnp.inf)
    ibig = jnp.int32(2 ** 30)

    def tile_body(t, carry):
        bv, bi = carry
        c0 = t * CB
        colsT = ptsT_ref[:, pl.ds(c0, CB)]
        batch_c = bcol_ref[:, pl.ds(c0, CB)]
        colsq = jnp.sum(colsT * colsT, axis=0, keepdims=True)
        score = colsq - 2.0 * jnp.dot(rows, colsT,
                                      preferred_element_type=jnp.float32)
        col_ids = lax.broadcasted_iota(jnp.int32, (KRB, CB), 1) + c0
        valid = (batch_r == batch_c) & (col_ids != row_ids)
        score = jnp.where(valid, score, inf)
        comb_v = jnp.concatenate([bv, score], axis=1)
        comb_i = jnp.concatenate([bi, col_ids], axis=1)
        nv, ni = [], []
        for _ in range(KNN):
            m = jnp.min(comb_v, axis=1, keepdims=True)
            ismin = comb_v == m
            sel = jnp.min(jnp.where(ismin, comb_i, ibig), axis=1,
                          keepdims=True)
            nv.append(m)
            ni.append(sel)
            comb_v = jnp.where(ismin & (comb_i == sel), inf, comb_v)
        return (jnp.concatenate(nv, axis=1), jnp.concatenate(ni, axis=1))

    init = (jnp.full((KRB, KNN), inf, jnp.float32),
            jnp.full((KRB, KNN), ibig, jnp.int32))
    _, bi = lax.fori_loop(lo, hi, tile_body, init)
    nbr_ref[...] = bi


def _pre_kernel(n_valid, f_ref, sc_ref, sh_ref, w1d_ref, w1b_ref, ws_ref,
                p_ref, q_ref, s_ref, st_ref):
    b = pl.program_id(0)

    @pl.when(b == 0)
    def _():
        st_ref[...] = jnp.zeros_like(st_ref)

    t = f_ref[...] * sc_ref[...] + sh_ref[...]
    p_ref[...] = jnp.dot(t, w1d_ref[...], preferred_element_type=jnp.float32)
    q_ref[...] = jnp.dot(t, w1b_ref[...], preferred_element_type=jnp.float32)
    s = jnp.dot(t, ws_ref[...], preferred_element_type=jnp.float32)
    s_ref[...] = s
    rid = lax.broadcasted_iota(jnp.int32, (1, s.shape[0]), 1) + b * s.shape[0]
    m = (rid < n_valid).astype(s.dtype)
    st_ref[0:1, :] += jnp.dot(m, s, preferred_element_type=jnp.float32)
    st_ref[1:2, :] += jnp.dot(m, s * s, preferred_element_type=jnp.float32)


def _edge_stats_kernel(n_valid, qg_ref, p_ref, st_ref):
    b = pl.program_id(0)

    @pl.when(b == 0)
    def _():
        st_ref[...] = jnp.zeros_like(st_ref)

    pb = p_ref[...]
    rid = lax.broadcasted_iota(jnp.int32, (1, pb.shape[0]), 1) + b * pb.shape[0]
    m = (rid < n_valid).astype(jnp.float32)
    sa = jnp.zeros((1, pb.shape[1]), jnp.float32)
    qa = jnp.zeros((1, pb.shape[1]), jnp.float32)
    for j in range(KNN):
        y = qg_ref[j] + pb
        sa = sa + jnp.dot(m, y, preferred_element_type=jnp.float32)
        qa = qa + jnp.dot(m, y * y, preferred_element_type=jnp.float32)
    st_ref[0:1, :] += sa
    st_ref[1:2, :] += qa


def _edge_mm1_kernel(n_valid, qg_ref, p_ref, mu_ref, w_ref, out_ref, st_ref):
    b = pl.program_id(0)

    @pl.when(b == 0)
    def _():
        st_ref[...] = jnp.zeros_like(st_ref)

    pb = p_ref[...] - mu_ref[...]
    rid = lax.broadcasted_iota(jnp.int32, (1, pb.shape[0]), 1) + b * pb.shape[0]
    m = (rid < n_valid).astype(jnp.float32)
    sa = jnp.zeros((1, w_ref.shape[1]), jnp.float32)
    qa = jnp.zeros((1, w_ref.shape[1]), jnp.float32)
    for j in range(KNN):
        h = jnp.maximum(qg_ref[j] + pb, 0.0)
        y2 = jnp.dot(h, w_ref[...], preferred_element_type=jnp.float32)
        out_ref[j] = y2
        sa = sa + jnp.dot(m, y2, preferred_element_type=jnp.float32)
        qa = qa + jnp.dot(m, y2 * y2, preferred_element_type=jnp.float32)
    st_ref[0:1, :] += sa
    st_ref[1:2, :] += qa


def _edge_mm2_kernel(n_valid, y_ref, mu_ref, w_ref, out_ref, st_ref):
    b = pl.program_id(0)

    @pl.when(b == 0)
    def _():
        st_ref[...] = jnp.zeros_like(st_ref)

    nrow = y_ref.shape[1]
    rid = lax.broadcasted_iota(jnp.int32, (1, nrow), 1) + b * nrow
    m = (rid < n_valid).astype(jnp.float32)
    mu = mu_ref[...]
    sa = jnp.zeros((1, w_ref.shape[1]), jnp.float32)
    qa = jnp.zeros((1, w_ref.shape[1]), jnp.float32)
    for j in range(KNN):
        h = jnp.maximum(y_ref[j] - mu, 0.0)
        y2 = jnp.dot(h, w_ref[...], preferred_element_type=jnp.float32)
        out_ref[j] = y2
        sa = sa + jnp.dot(m, y2, preferred_element_type=jnp.float32)
        qa = qa + jnp.dot(m, y2 * y2, preferred_element_type=jnp.float32)
    st_ref[0:1, :] += sa
    st_ref[1:2, :] += qa


def _aggr_kernel(y3_ref, mu3_ref, sc3_ref, s_ref, scs_ref, shs_ref, o_ref):
    mu3 = mu3_ref[...]
    acc = jnp.maximum(y3_ref[0] - mu3, 0.0)
    for j in range(1, KNN):
        acc = jnp.maximum(acc, y3_ref[j] - mu3)
    acc = jnp.maximum(acc, 0.0) * sc3_ref[...]
    skip = s_ref[...] * scs_ref[...] + shs_ref[...]
    o_ref[...] = jnp.maximum(acc + skip, 0.0)


def _pool_kernel(f_ref, bcol_ref, o_ref):
    b = pl.program_id(0)

    @pl.when(b == 0)
    def _():
        o_ref[...] = jnp.zeros_like(o_ref)

    fb = f_ref[...]
    bt = bcol_ref[...]
    seg = lax.broadcasted_iota(jnp.int32, (NG, fb.shape[0]), 0)
    onehot = (seg == bt).astype(jnp.float32)
    o_ref[...] += jnp.dot(onehot, fb, preferred_element_type=jnp.float32)


def _head_kernel(ncls, ps_ref, ic_ref, fcw_ref, fcb_ref, ow_ref, ob_ref,
                 o_ref):
    pooled = ps_ref[...] * ic_ref[...]
    h = jnp.maximum(
        jnp.dot(pooled, fcw_ref[...], preferred_element_type=jnp.float32)
        + fcb_ref[...], 0.0)
    lg = jnp.dot(h, ow_ref[...], preferred_element_type=jnp.float32) \
        + ob_ref[...]
    lane = lax.broadcasted_iota(jnp.int32, lg.shape, 1)
    lg = jnp.where(lane < ncls, lg, -jnp.inf)
    mx = jnp.max(lg, axis=1, keepdims=True)
    e = jnp.exp(lg - mx)
    o_ref[...] = e / jnp.sum(e, axis=1, keepdims=True)


def _gather_rows(q, idx):
    e = idx.shape[0]
    c = q.shape[1]
    info = plsc.get_sparse_core_info()
    nw = info.num_cores * info.num_subcores
    chunk = 128
    per_w = e // nw
    nch = per_w // chunk
    mesh = plsc.VectorSubcoreMesh(core_axis_name="c", subcore_axis_name="s")

    @functools.partial(
        pl.kernel,
        out_type=jax.ShapeDtypeStruct((e, c), jnp.float32),
        mesh=mesh,
        scratch_types=[
            pltpu.VMEM((chunk,), jnp.int32),
            pltpu.VMEM((chunk,), jnp.int32),
            pltpu.VMEM((chunk, c), jnp.float32),
            pltpu.VMEM((chunk, c), jnp.float32),
            pltpu.SemaphoreType.DMA,
            pltpu.SemaphoreType.DMA,
            pltpu.SemaphoreType.DMA,
            pltpu.SemaphoreType.DMA,
        ],
    )
    def gk(q_hbm, idx_hbm, out_hbm, i0, i1, r0, r1, g0, g1, s0, s1):
        wid = lax.axis_index("s") * info.num_cores + lax.axis_index("c")
        base = wid * per_w
        idx_v = (i0, i1)
        rows_v = (r0, r1)
        gsem = (g0, g1)
        ssem = (s0, s1)

        def body(t2, carry):
            offs = [base + (t2 * 2 + b) * chunk for b in range(2)]
            gets = []
            for b in range(2):
                pltpu.sync_copy(idx_hbm.at[pl.ds(offs[b], chunk)], idx_v[b])
                gets.append(pltpu.async_copy(q_hbm.at[idx_v[b]], rows_v[b],
                                             gsem[b]))
            puts = []
            for b in range(2):
                gets[b].wait()
                puts.append(pltpu.async_copy(
                    rows_v[b], out_hbm.at[pl.ds(offs[b], chunk)], ssem[b]))
            for b in range(2):
                puts[b].wait()
            return carry

        lax.fori_loop(0, nch // 2, body, 0)

    return gk(q, idx)


def _bn_coeffs(st, g, b, cnt):
    s = st[0:1, :]
    ss = st[1:2, :]
    mu = s / cnt
    var = ss / cnt - mu * mu
    inv = g.reshape(1, -1) / jnp.sqrt(var + EPS)
    return inv, b.reshape(1, -1) - mu * inv


def _bn_mu_inv(st, g, b, cnt):
    mu = st[0:1, :] / cnt
    var = st[1:2, :] / cnt - mu * mu
    inv = g.reshape(1, -1) / jnp.sqrt(var + EPS)
    return mu - b.reshape(1, -1) / inv, inv


def _colstats(xp, n):
    npad, d = xp.shape
    return _pallas_call(
        functools.partial(_colstats_kernel, n),
        grid=(npad // RB,),
        in_specs=[pl.BlockSpec((RB, d), lambda i: (i, 0))],
        out_specs=pl.BlockSpec((2, d), lambda i: (0, 0)),
        out_shape=jax.ShapeDtypeStruct((2, d), jnp.float32),
    )(xp)


def _knn(bounds, pts, ptsT, brow, bcol):
    npad, d = pts.shape
    return _pallas_call(
        _knn_kernel,
        grid=(npad // KRB,),
        in_specs=[
            pl.BlockSpec(memory_space=pltpu.SMEM),
            pl.BlockSpec((npad, d), lambda i: (0, 0)),
            pl.BlockSpec((d, npad), lambda i: (0, 0)),
            pl.BlockSpec((npad, 1), lambda i: (0, 0)),
            pl.BlockSpec((1, npad), lambda i: (0, 0)),
        ],
        out_specs=pl.BlockSpec((KRB, KNN), lambda i: (i, 0)),
        out_shape=jax.ShapeDtypeStruct((npad, KNN), jnp.int32),
    )(bounds, pts, ptsT, brow, bcol)


def _pre(fts, sc0, sh0, w1d, w1b, ws, n):
    npad, d = fts.shape
    c = w1d.shape[1]
    cq = w1b.shape[1]
    cs = ws.shape[1]
    return _pallas_call(
        functools.partial(_pre_kernel, n),
        grid=(npad // RB,),
        in_specs=[
            pl.BlockSpec((RB, d), lambda i: (i, 0)),
            pl.BlockSpec((1, d), lambda i: (0, 0)),
            pl.BlockSpec((1, d), lambda i: (0, 0)),
            pl.BlockSpec((d, c), lambda i: (0, 0)),
            pl.BlockSpec((d, cq), lambda i: (0, 0)),
            pl.BlockSpec((d, cs), lambda i: (0, 0)),
        ],
        out_specs=[
            pl.BlockSpec((RB, c), lambda i: (i, 0)),
            pl.BlockSpec((RB, cq), lambda i: (i, 0)),
            pl.BlockSpec((RB, cs), lambda i: (i, 0)),
            pl.BlockSpec((2, cs), lambda i: (0, 0)),
        ],
        out_shape=[
            jax.ShapeDtypeStruct((npad, c), jnp.float32),
            jax.ShapeDtypeStruct((npad, cq), jnp.float32),
            jax.ShapeDtypeStruct((npad, cs), jnp.float32),
            jax.ShapeDtypeStruct((2, cs), jnp.float32),
        ],
    )(fts, sc0, sh0, w1d, w1b, ws)


def _edge_stats(qg3, p, n):
    _, npad, _ = qg3.shape
    c = p.shape[1]
    return _pallas_call(
        functools.partial(_edge_stats_kernel, n),
        grid=(npad // EB,),
        in_specs=[
            pl.BlockSpec((KNN, EB, c), lambda i: (0, i, 0)),
            pl.BlockSpec((EB, c), lambda i: (i, 0)),
        ],
        out_specs=pl.BlockSpec((2, c), lambda i: (0, 0)),
        out_shape=jax.ShapeDtypeStruct((2, c), jnp.float32),
    )(qg3, p)


def _edge_mm1(qg3, p, mu, w, n):
    _, npad, _ = qg3.shape
    c = p.shape[1]
    c2 = w.shape[1]
    return _pallas_call(
        functools.partial(_edge_mm1_kernel, n),
        grid=(npad // EB,),
        in_specs=[
            pl.BlockSpec((KNN, EB, c), lambda i: (0, i, 0)),
            pl.BlockSpec((EB, c), lambda i: (i, 0)),
            pl.BlockSpec((1, c), lambda i: (0, 0)),
            pl.BlockSpec((c, c2), lambda i: (0, 0)),
        ],
        out_specs=[
            pl.BlockSpec((KNN, EB, c2), lambda i: (0, i, 0)),
            pl.BlockSpec((2, c2), lambda i: (0, 0)),
        ],
        out_shape=[
            jax.ShapeDtypeStruct((KNN, npad, c2), jnp.float32),
            jax.ShapeDtypeStruct((2, c2), jnp.float32),
        ],
    )(qg3, p, mu, w)


def _edge_mm2(y, mu, w, n):
    _, npad, c = y.shape
    c2 = w.shape[1]
    return _pallas_call(
        functools.partial(_edge_mm2_kernel, n),
        grid=(npad // EB,),
        in_specs=[
            pl.BlockSpec((KNN, EB, c), lambda i: (0, i, 0)),
            pl.BlockSpec((1, c), lambda i: (0, 0)),
            pl.BlockSpec((c, c2), lambda i: (0, 0)),
        ],
        out_specs=[
            pl.BlockSpec((KNN, EB, c2), lambda i: (0, i, 0)),
            pl.BlockSpec((2, c2), lambda i: (0, 0)),
        ],
        out_shape=[
            jax.ShapeDtypeStruct((KNN, npad, c2), jnp.float32),
            jax.ShapeDtypeStruct((2, c2), jnp.float32),
        ],
    )(y, mu, w)


def _aggr(y3, mu3, sc3, s, scs, shs):
    _, npad, c = y3.shape
    return _pallas_call(
        _aggr_kernel,
        grid=(npad // RB,),
        in_specs=[
            pl.BlockSpec((KNN, RB, c), lambda i: (0, i, 0)),
            pl.BlockSpec((1, c), lambda i: (0, 0)),
            pl.BlockSpec((1, c), lambda i: (0, 0)),
            pl.BlockSpec((RB, c), lambda i: (i, 0)),
            pl.BlockSpec((1, c), lambda i: (0, 0)),
            pl.BlockSpec((1, c), lambda i: (0, 0)),
        ],
        out_specs=pl.BlockSpec((RB, c), lambda i: (i, 0)),
        out_shape=jax.ShapeDtypeStruct((npad, c), jnp.float32),
    )(y3, mu3, sc3, s, scs, shs)


def _pool(fts, bcol):
    npad, c = fts.shape
    return _pallas_call(
        _pool_kernel,
        grid=(npad // RB,),
        in_specs=[
            pl.BlockSpec((RB, c), lambda i: (i, 0)),
            pl.BlockSpec((1, RB), lambda i: (0, i)),
        ],
        out_specs=pl.BlockSpec((NG, c), lambda i: (0, 0)),
        out_shape=jax.ShapeDtypeStruct((NG, c), jnp.float32),
    )(fts, bcol)


def _head(ps, ic, fcw, fcb, ow, ob, ncls):
    c = ps.shape[1]
    cf = fcw.shape[1]
    co = ow.shape[1]
    return _pallas_call(
        functools.partial(_head_kernel, ncls),
        in_specs=[
            pl.BlockSpec((NG, c), lambda: (0, 0)),
            pl.BlockSpec((NG, 1), lambda: (0, 0)),
            pl.BlockSpec((c, cf), lambda: (0, 0)),
            pl.BlockSpec((1, cf), lambda: (0, 0)),
            pl.BlockSpec((cf, co), lambda: (0, 0)),
            pl.BlockSpec((1, co), lambda: (0, 0)),
        ],
        out_specs=pl.BlockSpec((NG, co), lambda: (0, 0)),
        out_shape=jax.ShapeDtypeStruct((NG, co), jnp.float32),
    )(ps, ic, fcw, fcb, ow, ob)


def kernel(x, batch, params):
    n, din = x.shape
    npad = ((n + RB - 1) // RB) * RB
    pad = npad - n
    batch = batch.astype(jnp.int32)
    xp = jnp.pad(x, ((0, pad), (0, 0)))
    bp = jnp.pad(batch, (0, pad), constant_values=-1)
    brow = bp.reshape(npad, 1)
    bcol = bp.reshape(1, npad)

    gids = jnp.arange(NG, dtype=jnp.int32)
    seg_st = jnp.searchsorted(batch, gids, side='left').astype(jnp.int32)
    seg_en = jnp.searchsorted(batch, gids, side='right').astype(jnp.int32)

    nblk = npad // KRB
    first = jnp.minimum(jnp.arange(nblk, dtype=jnp.int32) * KRB, n - 1)
    last = jnp.minimum(first + KRB - 1, n - 1)
    cs = seg_st[batch[first]]
    ce = seg_en[batch[last]]
    bounds = jnp.stack([cs // CB, (ce + CB - 1) // CB], axis=1)

    st0 = _colstats(xp, n)
    scf, shf = _bn_coeffs(st0, params['bn_g'], params['bn_b'], float(n))

    fts = xp
    pts = jnp.pad(x[:, :3], ((0, pad), (0, 5)))
    ptsT = pts.T
    ecnt = float(n * KNN)

    for lp in params['convs']:
        d = fts.shape[1]
        w1 = lp['W1']
        w1d = (w1[:, :d] - w1[:, d:]).T
        w1b = w1[:, d:].T
        w2 = lp['W2'].T
        w3 = lp['W3'].T
        ws = lp['Ws'].T
        c = w1d.shape[1]

        cq = max(c, 128)
        w1dp = jnp.pad(w1d, ((0, 0), (0, cq - c)))
        w1bp = jnp.pad(w1b, ((0, 0), (0, cq - c)))
        w2p = jnp.pad(w2, ((0, cq - c), (0, 0)))

        nbr = _knn(bounds, pts, ptsT, brow, bcol)
        p, q, s, st_s = _pre(fts, scf, shf, w1dp, w1bp, ws, n)
        idx = jnp.where(nbr < n, nbr, 0).T.reshape(KNN * npad)
        qg3 = _gather_rows(q, idx).reshape(KNN, npad, cq)

        st1 = _edge_stats(qg3, p, n)
        mu1, inv1 = _bn_mu_inv(st1[:, :c], lp['g1'], lp['b1'], ecnt)
        mu1p = jnp.pad(mu1, ((0, 0), (0, cq - c)))
        w2f = w2p * jnp.pad(inv1, ((0, 0), (0, cq - c))).reshape(cq, 1)
        y2, st2 = _edge_mm1(qg3, p, mu1p, w2f, n)
        mu2, inv2 = _bn_mu_inv(st2, lp['g2'], lp['b2'], ecnt)
        w3f = w3 * inv2.reshape(-1, 1)
        y3, st3 = _edge_mm2(y2, mu2, w3f, n)
        mu3, inv3 = _bn_mu_inv(st3, lp['g3'], lp['b3'], ecnt)
        scs, shs = _bn_coeffs(st_s, lp['gs'], lp['bs'], float(n))
        fts = _aggr(y3, mu3, inv3, s, scs, shs)

        cn = fts.shape[1]
        scf = jnp.ones((1, cn), jnp.float32)
        shf = jnp.zeros((1, cn), jnp.float32)
        pts = fts
        ptsT = fts.T

    ps = _pool(fts, bcol)
    cnts = (seg_en - seg_st).astype(jnp.float32)
    ic = (1.0 / jnp.maximum(cnts, 1.0)).reshape(NG, 1)

    ncls = params['out_W'].shape[0]
    ow = jnp.pad(params['out_W'].T, ((0, 0), (0, 128 - ncls)))
    ob = jnp.pad(params['out_b'].reshape(1, -1), ((0, 0), (0, 128 - ncls)))
    probs = _head(ps, ic, params['fc_W'].T,
                  params['fc_b'].reshape(1, -1), ow, ob, ncls)
    return probs[:, :ncls]


# speedup vs baseline: 1.2229x; 1.0704x over previous
import functools

import jax
import jax.numpy as jnp
from jax import lax
from jax.experimental import pallas as pl
from jax.experimental.pallas import tpu as pltpu
from jax.experimental.pallas import tpu_sc as plsc

NG = 64
KNN = 16
EPS = 1e-5
RB = 256
CB = 256
EB = 128
KRB = 256

_pallas_call = pl.pallas_call


def _colstats_kernel(n_valid, x_ref, st_ref):
    b = pl.program_id(0)

    @pl.when(b == 0)
    def _():
        st_ref[...] = jnp.zeros_like(st_ref)

    xb = x_ref[...]
    rid = lax.broadcasted_iota(jnp.int32, (1, xb.shape[0]), 1) + b * xb.shape[0]
    m = (rid < n_valid).astype(xb.dtype)
    st_ref[0:1, :] += jnp.dot(m, xb, preferred_element_type=jnp.float32)
    st_ref[1:2, :] += jnp.dot(m, xb * xb, preferred_element_type=jnp.float32)


def _knn_kernel(bounds_ref, pts_ref, ptsT_ref, brow_ref, bcol_ref, nbr_ref):
    b = pl.program_id(0)
    r0 = b * KRB
    rows = pts_ref[pl.ds(r0, KRB), :]
    batch_r = brow_ref[pl.ds(r0, KRB), :]
    row_ids = lax.broadcasted_iota(jnp.int32, (KRB, 1), 0) + r0
    lo = bounds_ref[b, 0]
    hi = bounds_ref[b, 1]
    inf = jnp.float32(jnp.inf)
    ibig = jnp.int32(2 ** 30)
    imax = jnp.int32(0x7FFFFFFF)
    rowsq = jnp.sum(rows * rows, axis=1, keepdims=True)
    lane16 = lax.broadcasted_iota(jnp.int32, (KRB, KNN), 1)

    def tile_body(t, carry):
        bkey, bgid = carry
        c0 = t * CB
        colsT = ptsT_ref[:, pl.ds(c0, CB)]
        batch_c = bcol_ref[:, pl.ds(c0, CB)]
        colsq = jnp.sum(colsT * colsT, axis=0, keepdims=True)
        score = colsq - 2.0 * jnp.dot(rows, colsT,
                                      preferred_element_type=jnp.float32)
        col_ids = lax.broadcasted_iota(jnp.int32, (KRB, CB), 1) + c0
        valid = (batch_r == batch_c) & (col_ids != row_ids)
        d2 = jnp.where(valid, jnp.maximum(score + rowsq, 0.0), inf)
        key = lax.bitcast_convert_type(d2, jnp.int32)
        key = jnp.bitwise_or(jnp.bitwise_and(key, jnp.int32(~0x1FF)),
                             col_ids - c0)
        comb = jnp.concatenate([bkey, key], axis=1)
        ms = []
        for _ in range(KNN):
            m = jnp.min(comb, axis=1, keepdims=True)
            comb = jnp.where(comb == m, imax, comb)
            ms.append(m)
        nk, ng = [], []
        for t16 in range(KNN):
            m = ms[t16]
            slot = jnp.bitwise_and(m, jnp.int32(0x1FF))
            gid_best = jnp.min(jnp.where(lane16 + 256 == slot, bgid, ibig),
                               axis=1, keepdims=True)
            gid = jnp.where(slot < 256, c0 + slot, gid_best)
            nk.append(jnp.bitwise_or(jnp.bitwise_and(m, jnp.int32(~0x1FF)),
                                     jnp.int32(256 + t16)))
            ng.append(gid)
        return (jnp.concatenate(nk, axis=1), jnp.concatenate(ng, axis=1))

    init = (jnp.int32(0x7F800000 | 256) + lane16,
            jnp.zeros((KRB, KNN), jnp.int32))
    _, bi = lax.fori_loop(lo, hi, tile_body, init)
    nbr_ref[...] = bi


def _pre_kernel(n_valid, f_ref, sc_ref, sh_ref, w1d_ref, w1b_ref, ws_ref,
                p_ref, q_ref, s_ref, st_ref):
    b = pl.program_id(0)

    @pl.when(b == 0)
    def _():
        st_ref[...] = jnp.zeros_like(st_ref)

    t = f_ref[...] * sc_ref[...] + sh_ref[...]
    p_ref[...] = jnp.dot(t, w1d_ref[...], preferred_element_type=jnp.float32)
    q_ref[...] = jnp.dot(t, w1b_ref[...], preferred_element_type=jnp.float32)
    s = jnp.dot(t, ws_ref[...], preferred_element_type=jnp.float32)
    s_ref[...] = s
    rid = lax.broadcasted_iota(jnp.int32, (1, s.shape[0]), 1) + b * s.shape[0]
    m = (rid < n_valid).astype(s.dtype)
    st_ref[0:1, :] += jnp.dot(m, s, preferred_element_type=jnp.float32)
    st_ref[1:2, :] += jnp.dot(m, s * s, preferred_element_type=jnp.float32)


def _edge_stats_kernel(n_valid, qg_ref, p_ref, st_ref):
    b = pl.program_id(0)

    @pl.when(b == 0)
    def _():
        st_ref[...] = jnp.zeros_like(st_ref)

    pb = p_ref[...]
    rid = lax.broadcasted_iota(jnp.int32, (1, pb.shape[0]), 1) + b * pb.shape[0]
    m = (rid < n_valid).astype(jnp.float32)
    sa = jnp.zeros((1, pb.shape[1]), jnp.float32)
    qa = jnp.zeros((1, pb.shape[1]), jnp.float32)
    for j in range(KNN):
        y = qg_ref[j] + pb
        sa = sa + jnp.dot(m, y, preferred_element_type=jnp.float32)
        qa = qa + jnp.dot(m, y * y, preferred_element_type=jnp.float32)
    st_ref[0:1, :] += sa
    st_ref[1:2, :] += qa


def _edge_mm1_kernel(n_valid, qg_ref, p_ref, mu_ref, w_ref, out_ref, st_ref):
    b = pl.program_id(0)

    @pl.when(b == 0)
    def _():
        st_ref[...] = jnp.zeros_like(st_ref)

    pb = p_ref[...] - mu_ref[...]
    rid = lax.broadcasted_iota(jnp.int32, (1, pb.shape[0]), 1) + b * pb.shape[0]
    m = (rid < n_valid).astype(jnp.float32)
    sa = jnp.zeros((1, w_ref.shape[1]), jnp.float32)
    qa = jnp.zeros((1, w_ref.shape[1]), jnp.float32)
    for j in range(KNN):
        h = jnp.maximum(qg_ref[j] + pb, 0.0)
        y2 = jnp.dot(h, w_ref[...], preferred_element_type=jnp.float32)
        out_ref[j] = y2
        sa = sa + jnp.dot(m, y2, preferred_element_type=jnp.float32)
        qa = qa + jnp.dot(m, y2 * y2, preferred_element_type=jnp.float32)
    st_ref[0:1, :] += sa
    st_ref[1:2, :] += qa


def _edge_mm2_kernel(n_valid, y_ref, mu_ref, w_ref, out_ref, st_ref):
    b = pl.program_id(0)

    @pl.when(b == 0)
    def _():
        st_ref[...] = jnp.zeros_like(st_ref)

    nrow = y_ref.shape[1]
    rid = lax.broadcasted_iota(jnp.int32, (1, nrow), 1) + b * nrow
    m = (rid < n_valid).astype(jnp.float32)
    mu = mu_ref[...]
    sa = jnp.zeros((1, w_ref.shape[1]), jnp.float32)
    qa = jnp.zeros((1, w_ref.shape[1]), jnp.float32)
    for j in range(KNN):
        h = jnp.maximum(y_ref[j] - mu, 0.0)
        y2 = jnp.dot(h, w_ref[...], preferred_element_type=jnp.float32)
        out_ref[j] = y2
        sa = sa + jnp.dot(m, y2, preferred_element_type=jnp.float32)
        qa = qa + jnp.dot(m, y2 * y2, preferred_element_type=jnp.float32)
    st_ref[0:1, :] += sa
    st_ref[1:2, :] += qa


def _aggr_kernel(y3_ref, mu3_ref, sc3_ref, s_ref, scs_ref, shs_ref, o_ref):
    mu3 = mu3_ref[...]
    acc = jnp.maximum(y3_ref[0] - mu3, 0.0)
    for j in range(1, KNN):
        acc = jnp.maximum(acc, y3_ref[j] - mu3)
    acc = jnp.maximum(acc, 0.0) * sc3_ref[...]
    skip = s_ref[...] * scs_ref[...] + shs_ref[...]
    o_ref[...] = jnp.maximum(acc + skip, 0.0)


def _pool_kernel(f_ref, bcol_ref, o_ref):
    b = pl.program_id(0)

    @pl.when(b == 0)
    def _():
        o_ref[...] = jnp.zeros_like(o_ref)

    fb = f_ref[...]
    bt = bcol_ref[...]
    seg = lax.broadcasted_iota(jnp.int32, (NG, fb.shape[0]), 0)
    onehot = (seg == bt).astype(jnp.float32)
    o_ref[...] += jnp.dot(onehot, fb, preferred_element_type=jnp.float32)


def _head_kernel(ncls, ps_ref, ic_ref, fcw_ref, fcb_ref, ow_ref, ob_ref,
                 o_ref):
    pooled = ps_ref[...] * ic_ref[...]
    h = jnp.maximum(
        jnp.dot(pooled, fcw_ref[...], preferred_element_type=jnp.float32)
        + fcb_ref[...], 0.0)
    lg = jnp.dot(h, ow_ref[...], preferred_element_type=jnp.float32) \
        + ob_ref[...]
    lane = lax.broadcasted_iota(jnp.int32, lg.shape, 1)
    lg = jnp.where(lane < ncls, lg, -jnp.inf)
    mx = jnp.max(lg, axis=1, keepdims=True)
    e = jnp.exp(lg - mx)
    o_ref[...] = e / jnp.sum(e, axis=1, keepdims=True)


def _gather_rows(q, idx):
    e = idx.shape[0]
    c = q.shape[1]
    info = plsc.get_sparse_core_info()
    nw = info.num_cores * info.num_subcores
    chunk = 128
    per_w = e // nw
    nch = per_w // chunk
    mesh = plsc.VectorSubcoreMesh(core_axis_name="c", subcore_axis_name="s")

    @functools.partial(
        pl.kernel,
        out_type=jax.ShapeDtypeStruct((e, c), jnp.float32),
        mesh=mesh,
        scratch_types=[
            pltpu.VMEM((chunk,), jnp.int32),
            pltpu.VMEM((chunk,), jnp.int32),
            pltpu.VMEM((chunk, c), jnp.float32),
            pltpu.VMEM((chunk, c), jnp.float32),
            pltpu.SemaphoreType.DMA,
            pltpu.SemaphoreType.DMA,
            pltpu.SemaphoreType.DMA,
            pltpu.SemaphoreType.DMA,
        ],
    )
    def gk(q_hbm, idx_hbm, out_hbm, i0, i1, r0, r1, g0, g1, s0, s1):
        wid = lax.axis_index("s") * info.num_cores + lax.axis_index("c")
        base = wid * per_w
        idx_v = (i0, i1)
        rows_v = (r0, r1)
        gsem = (g0, g1)
        ssem = (s0, s1)

        def body(t2, carry):
            offs = [base + (t2 * 2 + b) * chunk for b in range(2)]
            gets = []
            for b in range(2):
                pltpu.sync_copy(idx_hbm.at[pl.ds(offs[b], chunk)], idx_v[b])
                gets.append(pltpu.async_copy(q_hbm.at[idx_v[b]], rows_v[b],
                                             gsem[b]))
            puts = []
            for b in range(2):
                gets[b].wait()
                puts.append(pltpu.async_copy(
                    rows_v[b], out_hbm.at[pl.ds(offs[b], chunk)], ssem[b]))
            for b in range(2):
                puts[b].wait()
            return carry

        lax.fori_loop(0, nch // 2, body, 0)

    return gk(q, idx)


def _bn_coeffs(st, g, b, cnt):
    s = st[0:1, :]
    ss = st[1:2, :]
    mu = s / cnt
    var = ss / cnt - mu * mu
    inv = g.reshape(1, -1) / jnp.sqrt(var + EPS)
    return inv, b.reshape(1, -1) - mu * inv


def _bn_mu_inv(st, g, b, cnt):
    mu = st[0:1, :] / cnt
    var = st[1:2, :] / cnt - mu * mu
    inv = g.reshape(1, -1) / jnp.sqrt(var + EPS)
    return mu - b.reshape(1, -1) / inv, inv


def _colstats(xp, n):
    npad, d = xp.shape
    return _pallas_call(
        functools.partial(_colstats_kernel, n),
        grid=(npad // RB,),
        in_specs=[pl.BlockSpec((RB, d), lambda i: (i, 0))],
        out_specs=pl.BlockSpec((2, d), lambda i: (0, 0)),
        out_shape=jax.ShapeDtypeStruct((2, d), jnp.float32),
    )(xp)


def _knn(bounds, pts, ptsT, brow, bcol):
    npad, d = pts.shape
    return _pallas_call(
        _knn_kernel,
        grid=(npad // KRB,),
        in_specs=[
            pl.BlockSpec(memory_space=pltpu.SMEM),
            pl.BlockSpec((npad, d), lambda i: (0, 0)),
            pl.BlockSpec((d, npad), lambda i: (0, 0)),
            pl.BlockSpec((npad, 1), lambda i: (0, 0)),
            pl.BlockSpec((1, npad), lambda i: (0, 0)),
        ],
        out_specs=pl.BlockSpec((KRB, KNN), lambda i: (i, 0)),
        out_shape=jax.ShapeDtypeStruct((npad, KNN), jnp.int32),
    )(bounds, pts, ptsT, brow, bcol)


def _pre(fts, sc0, sh0, w1d, w1b, ws, n):
    npad, d = fts.shape
    c = w1d.shape[1]
    cq = w1b.shape[1]
    cs = ws.shape[1]
    return _pallas_call(
        functools.partial(_pre_kernel, n),
        grid=(npad // RB,),
        in_specs=[
            pl.BlockSpec((RB, d), lambda i: (i, 0)),
            pl.BlockSpec((1, d), lambda i: (0, 0)),
            pl.BlockSpec((1, d), lambda i: (0, 0)),
            pl.BlockSpec((d, c), lambda i: (0, 0)),
            pl.BlockSpec((d, cq), lambda i: (0, 0)),
            pl.BlockSpec((d, cs), lambda i: (0, 0)),
        ],
        out_specs=[
            pl.BlockSpec((RB, c), lambda i: (i, 0)),
            pl.BlockSpec((RB, cq), lambda i: (i, 0)),
            pl.BlockSpec((RB, cs), lambda i: (i, 0)),
            pl.BlockSpec((2, cs), lambda i: (0, 0)),
        ],
        out_shape=[
            jax.ShapeDtypeStruct((npad, c), jnp.float32),
            jax.ShapeDtypeStruct((npad, cq), jnp.float32),
            jax.ShapeDtypeStruct((npad, cs), jnp.float32),
            jax.ShapeDtypeStruct((2, cs), jnp.float32),
        ],
    )(fts, sc0, sh0, w1d, w1b, ws)


def _edge_stats(qg3, p, n):
    _, npad, _ = qg3.shape
    c = p.shape[1]
    return _pallas_call(
        functools.partial(_edge_stats_kernel, n),
        grid=(npad // EB,),
        in_specs=[
            pl.BlockSpec((KNN, EB, c), lambda i: (0, i, 0)),
            pl.BlockSpec((EB, c), lambda i: (i, 0)),
        ],
        out_specs=pl.BlockSpec((2, c), lambda i: (0, 0)),
        out_shape=jax.ShapeDtypeStruct((2, c), jnp.float32),
    )(qg3, p)


def _edge_mm1(qg3, p, mu, w, n):
    _, npad, _ = qg3.shape
    c = p.shape[1]
    c2 = w.shape[1]
    return _pallas_call(
        functools.partial(_edge_mm1_kernel, n),
        grid=(npad // EB,),
        in_specs=[
            pl.BlockSpec((KNN, EB, c), lambda i: (0, i, 0)),
            pl.BlockSpec((EB, c), lambda i: (i, 0)),
            pl.BlockSpec((1, c), lambda i: (0, 0)),
            pl.BlockSpec((c, c2), lambda i: (0, 0)),
        ],
        out_specs=[
            pl.BlockSpec((KNN, EB, c2), lambda i: (0, i, 0)),
            pl.BlockSpec((2, c2), lambda i: (0, 0)),
        ],
        out_shape=[
            jax.ShapeDtypeStruct((KNN, npad, c2), jnp.float32),
            jax.ShapeDtypeStruct((2, c2), jnp.float32),
        ],
    )(qg3, p, mu, w)


def _edge_mm2(y, mu, w, n):
    _, npad, c = y.shape
    c2 = w.shape[1]
    return _pallas_call(
        functools.partial(_edge_mm2_kernel, n),
        grid=(npad // EB,),
        in_specs=[
            pl.BlockSpec((KNN, EB, c), lambda i: (0, i, 0)),
            pl.BlockSpec((1, c), lambda i: (0, 0)),
            pl.BlockSpec((c, c2), lambda i: (0, 0)),
        ],
        out_specs=[
            pl.BlockSpec((KNN, EB, c2), lambda i: (0, i, 0)),
            pl.BlockSpec((2, c2), lambda i: (0, 0)),
        ],
        out_shape=[
            jax.ShapeDtypeStruct((KNN, npad, c2), jnp.float32),
            jax.ShapeDtypeStruct((2, c2), jnp.float32),
        ],
    )(y, mu, w)


def _aggr(y3, mu3, sc3, s, scs, shs):
    _, npad, c = y3.shape
    return _pallas_call(
        _aggr_kernel,
        grid=(npad // RB,),
        in_specs=[
            pl.BlockSpec((KNN, RB, c), lambda i: (0, i, 0)),
            pl.BlockSpec((1, c), lambda i: (0, 0)),
            pl.BlockSpec((1, c), lambda i: (0, 0)),
            pl.BlockSpec((RB, c), lambda i: (i, 0)),
            pl.BlockSpec((1, c), lambda i: (0, 0)),
            pl.BlockSpec((1, c), lambda i: (0, 0)),
        ],
        out_specs=pl.BlockSpec((RB, c), lambda i: (i, 0)),
        out_shape=jax.ShapeDtypeStruct((npad, c), jnp.float32),
    )(y3, mu3, sc3, s, scs, shs)


def _pool(fts, bcol):
    npad, c = fts.shape
    return _pallas_call(
        _pool_kernel,
        grid=(npad // RB,),
        in_specs=[
            pl.BlockSpec((RB, c), lambda i: (i, 0)),
            pl.BlockSpec((1, RB), lambda i: (0, i)),
        ],
        out_specs=pl.BlockSpec((NG, c), lambda i: (0, 0)),
        out_shape=jax.ShapeDtypeStruct((NG, c), jnp.float32),
    )(fts, bcol)


def _head(ps, ic, fcw, fcb, ow, ob, ncls):
    c = ps.shape[1]
    cf = fcw.shape[1]
    co = ow.shape[1]
    return _pallas_call(
        functools.partial(_head_kernel, ncls),
        in_specs=[
            pl.BlockSpec((NG, c), lambda: (0, 0)),
            pl.BlockSpec((NG, 1), lambda: (0, 0)),
            pl.BlockSpec((c, cf), lambda: (0, 0)),
            pl.BlockSpec((1, cf), lambda: (0, 0)),
            pl.BlockSpec((cf, co), lambda: (0, 0)),
            pl.BlockSpec((1, co), lambda: (0, 0)),
        ],
        out_specs=pl.BlockSpec((NG, co), lambda: (0, 0)),
        out_shape=jax.ShapeDtypeStruct((NG, co), jnp.float32),
    )(ps, ic, fcw, fcb, ow, ob)


def kernel(x, batch, params):
    n, din = x.shape
    npad = ((n + RB - 1) // RB) * RB
    pad = npad - n
    batch = batch.astype(jnp.int32)
    xp = jnp.pad(x, ((0, pad), (0, 0)))
    bp = jnp.pad(batch, (0, pad), constant_values=-1)
    brow = bp.reshape(npad, 1)
    bcol = bp.reshape(1, npad)

    gids = jnp.arange(NG, dtype=jnp.int32)
    seg_st = jnp.searchsorted(batch, gids, side='left').astype(jnp.int32)
    seg_en = jnp.searchsorted(batch, gids, side='right').astype(jnp.int32)

    nblk = npad // KRB
    first = jnp.minimum(jnp.arange(nblk, dtype=jnp.int32) * KRB, n - 1)
    last = jnp.minimum(first + KRB - 1, n - 1)
    cs = seg_st[batch[first]]
    ce = seg_en[batch[last]]
    bounds = jnp.stack([cs // CB, (ce + CB - 1) // CB], axis=1)

    st0 = _colstats(xp, n)
    scf, shf = _bn_coeffs(st0, params['bn_g'], params['bn_b'], float(n))

    fts = xp
    pts = jnp.pad(x[:, :3], ((0, pad), (0, 5)))
    ptsT = pts.T
    ecnt = float(n * KNN)

    for lp in params['convs']:
        d = fts.shape[1]
        w1 = lp['W1']
        w1d = (w1[:, :d] - w1[:, d:]).T
        w1b = w1[:, d:].T
        w2 = lp['W2'].T
        w3 = lp['W3'].T
        ws = lp['Ws'].T
        c = w1d.shape[1]

        cq = max(c, 128)
        w1dp = jnp.pad(w1d, ((0, 0), (0, cq - c)))
        w1bp = jnp.pad(w1b, ((0, 0), (0, cq - c)))
        w2p = jnp.pad(w2, ((0, cq - c), (0, 0)))

        nbr = _knn(bounds, pts, ptsT, brow, bcol)
        p, q, s, st_s = _pre(fts, scf, shf, w1dp, w1bp, ws, n)
        idx = jnp.where(nbr < n, nbr, 0).T.reshape(KNN * npad)
        qg3 = _gather_rows(q, idx).reshape(KNN, npad, cq)

        st1 = _edge_stats(qg3, p, n)
        mu1, inv1 = _bn_mu_inv(st1[:, :c], lp['g1'], lp['b1'], ecnt)
        mu1p = jnp.pad(mu1, ((0, 0), (0, cq - c)))
        w2f = w2p * jnp.pad(inv1, ((0, 0), (0, cq - c))).reshape(cq, 1)
        y2, st2 = _edge_mm1(qg3, p, mu1p, w2f, n)
        mu2, inv2 = _bn_mu_inv(st2, lp['g2'], lp['b2'], ecnt)
        w3f = w3 * inv2.reshape(-1, 1)
        y3, st3 = _edge_mm2(y2, mu2, w3f, n)
        mu3, inv3 = _bn_mu_inv(st3, lp['g3'], lp['b3'], ecnt)
        scs, shs = _bn_coeffs(st_s, lp['gs'], lp['bs'], float(n))
        fts = _aggr(y3, mu3, inv3, s, scs, shs)

        cn = fts.shape[1]
        scf = jnp.ones((1, cn), jnp.float32)
        shf = jnp.zeros((1, cn), jnp.float32)
        pts = fts
        ptsT = fts.T

    ps = _pool(fts, bcol)
    cnts = (seg_en - seg_st).astype(jnp.float32)
    ic = (1.0 / jnp.maximum(cnts, 1.0)).reshape(NG, 1)

    ncls = params['out_W'].shape[0]
    ow = jnp.pad(params['out_W'].T, ((0, 0), (0, 128 - ncls)))
    ob = jnp.pad(params['out_b'].reshape(1, -1), ((0, 0), (0, 128 - ncls)))
    probs = _head(ps, ic, params['fc_W'].T,
                  params['fc_b'].reshape(1, -1), ow, ob, ncls)
    return probs[:, :ncls]
